# Initial kernel scaffold; baseline (speedup 1.0000x reference)
import functools

import jax
import jax.numpy as jnp
from jax import lax
from jax.experimental import pallas as pl
from jax.experimental.pallas import tpu as pltpu
from jax.experimental.pallas import tpu_sc as plsc

DIM = 64
RVQ = 4
NC = 2
NS = 16
NW = NC * NS
GCHUNK = 128
ROWS_PER_ITER = 128
IDX_PER_ITER = ROWS_PER_ITER * RVQ


def _build(n_rows, interpret=False):
    assert n_rows % (NW * ROWS_PER_ITER) == 0
    iters_per_w = n_rows // (NW * ROWS_PER_ITER)
    mesh = plsc.VectorSubcoreMesh(core_axis_name="c", subcore_axis_name="s")

    @functools.partial(
        pl.kernel,
        out_type=jax.ShapeDtypeStruct((n_rows, DIM), jnp.float32),
        mesh=mesh,
        scratch_types=[
            pltpu.VMEM((RVQ, GCHUNK), jnp.int32),
            pltpu.VMEM((RVQ, GCHUNK, DIM), jnp.float32),
            pltpu.VMEM((ROWS_PER_ITER, DIM), jnp.float32),
            pltpu.SemaphoreType.DMA,
        ],
        interpret=interpret,
    )
    def k(xc_hbm, emb_hbm, out_hbm, idx_v, g_v, out_v, sem):
        wid = lax.axis_index("s") * NC + lax.axis_index("c")

        def chunk(i, carry):
            ch = wid * iters_per_w + i
            pltpu.sync_copy(xc_hbm.at[ch], idx_v)
            cps = [
                pltpu.async_copy(emb_hbm.at[idx_v.at[kk]], g_v.at[kk], sem)
                for kk in range(RVQ)
            ]
            for cp in cps:
                cp.wait()

            def rows(t, c2):
                for kk in range(RVQ):
                    for d in range(DIM // 16):
                        sl = pl.ds(d * 16, 16)
                        acc = (
                            g_v[kk, 4 * t, sl]
                            + g_v[kk, 4 * t + 1, sl]
                            + g_v[kk, 4 * t + 2, sl]
                            + g_v[kk, 4 * t + 3, sl]
                        )
                        out_v[kk * (GCHUNK // RVQ) + t, sl] = acc
                return c2

            lax.fori_loop(0, GCHUNK // RVQ, rows, 0)
            pltpu.sync_copy(out_v, out_hbm.at[pl.ds(ch * ROWS_PER_ITER, ROWS_PER_ITER)])
            return carry

        lax.fori_loop(0, iters_per_w, chunk, 0)

    return k


def kernel(x, emb):
    B, W = x.shape
    L = W // RVQ
    n_rows = B * L
    xc = x.reshape(n_rows // ROWS_PER_ITER, RVQ, GCHUNK)
    out = _build(n_rows)(xc, emb)
    return out.reshape(B, L, DIM)


# speedup vs baseline: 15.5462x; 15.5462x over previous
import functools

import jax
import jax.numpy as jnp
from jax import lax
from jax.experimental import pallas as pl
from jax.experimental.pallas import tpu as pltpu
from jax.experimental.pallas import tpu_sc as plsc

DIM = 64
RVQ = 4
NC = 2
NS = 16
NW = NC * NS
GCHUNK = 128
ROWS_PER_ITER = 128


def _build(n_rows):
    assert n_rows % (NW * ROWS_PER_ITER) == 0
    niters = n_rows // (NW * ROWS_PER_ITER)
    assert niters % 2 == 0
    mesh = plsc.VectorSubcoreMesh(core_axis_name="c", subcore_axis_name="s")

    @functools.partial(
        pl.kernel,
        out_type=jax.ShapeDtypeStruct((n_rows, DIM), jnp.float32),
        mesh=mesh,
        scratch_types=[
            pltpu.VMEM((2, RVQ, GCHUNK), jnp.int32),
            pltpu.VMEM((2, RVQ, GCHUNK, DIM), jnp.float32),
            pltpu.VMEM((2, ROWS_PER_ITER, DIM), jnp.float32),
            pltpu.SemaphoreType.DMA,
            pltpu.SemaphoreType.DMA,
            pltpu.SemaphoreType.DMA,
            pltpu.SemaphoreType.DMA,
            pltpu.SemaphoreType.DMA,
            pltpu.SemaphoreType.DMA,
        ],
        compiler_params=pltpu.CompilerParams(use_tc_tiling_on_sc=False),
    )
    def k(xc_hbm, emb_hbm, out_hbm, idx_v, g_v, out_v,
          si0, si1, sg0, sg1, so0, so1):
        sem_i, sem_g, sem_o = (si0, si1), (sg0, sg1), (so0, so1)
        wid = lax.axis_index("s") * NC + lax.axis_index("c")
        base = wid * niters

        def fire_gathers(ch, b):
            for kk in range(RVQ):
                pltpu.make_async_copy(
                    emb_hbm.at[idx_v.at[b, kk]], g_v.at[b, kk], sem_g[b]
                ).start()

        def wait_gathers(b):
            for kk in range(RVQ):
                pltpu.make_async_copy(
                    emb_hbm.at[idx_v.at[b, kk]], g_v.at[b, kk], sem_g[b]
                ).wait()

        def fire_idx(ch, b):
            pltpu.make_async_copy(xc_hbm.at[ch], idx_v.at[b], sem_i[b]).start()

        def wait_idx(ch, b):
            pltpu.make_async_copy(xc_hbm.at[ch], idx_v.at[b], sem_i[b]).wait()

        def fire_out(ch, b):
            pltpu.make_async_copy(
                out_v.at[b],
                out_hbm.at[pl.ds(ch * ROWS_PER_ITER, ROWS_PER_ITER)],
                sem_o[b],
            ).start()

        def wait_out(ch, b):
            pltpu.make_async_copy(
                out_v.at[b],
                out_hbm.at[pl.ds(ch * ROWS_PER_ITER, ROWS_PER_ITER)],
                sem_o[b],
            ).wait()

        pltpu.sync_copy(xc_hbm.at[base], idx_v.at[0])
        fire_gathers(base, 0)
        fire_idx(base + 1, 1)

        def outer(j, carry):
            for b in range(2):
                i = 2 * j + b
                ch = base + i
                nb = 1 - b

                @pl.when(i + 1 < niters)
                def _():
                    wait_idx(ch + 1, nb)
                    fire_gathers(ch + 1, nb)

                wait_gathers(b)

                @pl.when(i + 2 < niters)
                def _():
                    fire_idx(ch + 2, b)

                @pl.when(i >= 2)
                def _():
                    wait_out(ch - 2, b)

                def rows(t, c2):
                    for kk in range(RVQ):
                        vals = [
                            [
                                g_v[b, kk, 4 * t + q, pl.ds(d * 16, 16)]
                                for q in range(RVQ)
                            ]
                            for d in range(DIM // 16)
                        ]
                        for d in range(DIM // 16):
                            v0, v1, v2, v3 = vals[d]
                            out_v[b, kk * (GCHUNK // RVQ) + t, pl.ds(d * 16, 16)] = (
                                (v0 + v1) + (v2 + v3)
                            )
                    return c2

                lax.fori_loop(0, GCHUNK // RVQ, rows, 0)
                fire_out(ch, b)
            return carry

        lax.fori_loop(0, niters // 2, outer, 0)
        wait_out(base + niters - 2, 0)
        wait_out(base + niters - 1, 1)

    return k


def kernel(x, emb):
    B, W = x.shape
    L = W // RVQ
    n_rows = B * L
    xc = x.reshape(n_rows // ROWS_PER_ITER, RVQ, GCHUNK)
    out = _build(n_rows)(xc, emb)
    return out.reshape(B, L, DIM)
